# baseline (speedup 1.0000x reference)
import jax
import jax.numpy as jnp
from jax import lax
from jax.experimental import pallas as pl
from jax.experimental.pallas import tpu as pltpu
from jax.experimental.pallas import tpu_sc as plsc

_B = 4096
_L = 50
_E = 32
_V = 100000

_NC = 2
_NS = 16
_NW = _NC * _NS
_G = 128
_NCHUNK = _B // _G
_LANES = 16
_NS16 = _G // _LANES


def _bow_body(xt_ref, embt_ref, bowt_ref, row_v, idx_v, out_v, sem0, sem1):
    wid = lax.axis_index("s") * _NC + lax.axis_index("c")
    inv_l = jnp.float32(1.0 / _L)
    sems = (sem0, sem1)

    row_cp = pltpu.async_copy(embt_ref.at[wid], row_v, sem0)

    cp0 = pltpu.async_copy(xt_ref.at[0], idx_v.at[0], sem1)
    row_cp.wait()
    cp0.wait()

    for c in range(_NCHUNK):
        if c + 1 < _NCHUNK:
            nxt = pltpu.async_copy(
                xt_ref.at[c + 1], idx_v.at[(c + 1) % 2], sems[(c + 1) % 2]
            )

        def strip(s, carry2, c=c):
            def tok(j, acc):
                iv = idx_v[c % 2, j, pl.ds(s * _LANES, _LANES)]
                return acc + plsc.load_gather(row_v, [iv])

            acc = lax.fori_loop(
                0, _L, tok, jnp.zeros((_LANES,), jnp.float32), unroll=25
            )
            out_v[pl.ds(c * _G + s * _LANES, _LANES)] = acc * inv_l
            return carry2

        lax.fori_loop(0, _NS16, strip, 0)
        if c + 1 < _NCHUNK:
            nxt.wait()

    pltpu.sync_copy(out_v, bowt_ref.at[wid])


def _bow_call(xt, embt):
    mesh = plsc.VectorSubcoreMesh(core_axis_name="c", subcore_axis_name="s")
    f = pl.kernel(
        _bow_body,
        out_type=jax.ShapeDtypeStruct((_NW, _B), jnp.float32),
        mesh=mesh,
        scratch_types=[
            pltpu.VMEM((_V,), jnp.float32),
            pltpu.VMEM((2, _L, _G), jnp.int32),
            pltpu.VMEM((_B,), jnp.float32),
            pltpu.SemaphoreType.DMA,
            pltpu.SemaphoreType.DMA,
        ],
        compiler_params=pltpu.CompilerParams(
            use_tc_tiling_on_sc=False, needs_layout_passes=False
        ),
    )
    return f(xt, embt)


_BV = 1024
_K = _E + 1


def _mm_body(wb_ref, bowt_ref, out_ref):
    out_ref[...] = lax.dot_general(
        wb_ref[...],
        bowt_ref[...],
        dimension_numbers=(((0,), (0,)), ((), ())),
        preferred_element_type=jnp.float32,
    )


def _mm_call(wb, bowt1):
    return pl.pallas_call(
        _mm_body,
        grid=(pl.cdiv(_V, _BV),),
        in_specs=[
            pl.BlockSpec((_K, _BV), lambda j: (0, j)),
            pl.BlockSpec((_K, _B), lambda j: (0, 0)),
        ],
        out_specs=pl.BlockSpec((_BV, _B), lambda j: (j, 0)),
        out_shape=jax.ShapeDtypeStruct((_V, _B), jnp.float32),
        compiler_params=pltpu.CompilerParams(
            dimension_semantics=("arbitrary",),
        ),
    )(wb, bowt1)


def kernel(X, emb_table, W, b):
    xt = X.astype(jnp.int32).T.reshape(_L, _NCHUNK, _G).transpose(1, 0, 2)
    bowt = _bow_call(xt, emb_table.T)
    wb = jnp.concatenate([W.T, b[None, :]], axis=0)
    bowt1 = jnp.concatenate([bowt, jnp.ones((1, _B), jnp.float32)], axis=0)
    return _mm_call(wb, bowt1).T


# speedup vs baseline: 1.0039x; 1.0039x over previous
import jax
import jax.numpy as jnp
from jax import lax
from jax.experimental import pallas as pl
from jax.experimental.pallas import tpu as pltpu
from jax.experimental.pallas import tpu_sc as plsc

_B = 4096
_L = 50
_E = 32
_V = 100000

_NC = 2
_NS = 16
_NW = _NC * _NS
_G = 128
_NCHUNK = _B // _G
_LANES = 16
_NS16 = _G // _LANES


def _bow_body(xt_ref, embt_ref, bowt_ref, row_v, idx_v, out_v, sem0, sem1):
    wid = lax.axis_index("s") * _NC + lax.axis_index("c")
    inv_l = jnp.float32(1.0 / _L)
    sems = (sem0, sem1)

    row_cp = pltpu.async_copy(embt_ref.at[wid], row_v, sem0)

    cp0 = pltpu.async_copy(xt_ref.at[0], idx_v.at[0], sem1)
    row_cp.wait()
    cp0.wait()

    for c in range(_NCHUNK):
        if c + 1 < _NCHUNK:
            nxt = pltpu.async_copy(
                xt_ref.at[c + 1], idx_v.at[(c + 1) % 2], sems[(c + 1) % 2]
            )

        def strip(s, carry2, c=c):
            def tok(j, acc):
                iv = idx_v[c % 2, j, pl.ds(s * _LANES, _LANES)]
                return acc + plsc.load_gather(row_v, [iv])

            acc = lax.fori_loop(
                0, _L, tok, jnp.zeros((_LANES,), jnp.float32), unroll=10
            )
            out_v[pl.ds(c * _G + s * _LANES, _LANES)] = acc * inv_l
            return carry2

        lax.fori_loop(0, _NS16, strip, 0)
        if c + 1 < _NCHUNK:
            nxt.wait()

    pltpu.sync_copy(out_v, bowt_ref.at[wid])


def _bow_call(xt, embt):
    mesh = plsc.VectorSubcoreMesh(core_axis_name="c", subcore_axis_name="s")
    f = pl.kernel(
        _bow_body,
        out_type=jax.ShapeDtypeStruct((_NW, _B), jnp.float32),
        mesh=mesh,
        scratch_types=[
            pltpu.VMEM((_V,), jnp.float32),
            pltpu.VMEM((2, _L, _G), jnp.int32),
            pltpu.VMEM((_B,), jnp.float32),
            pltpu.SemaphoreType.DMA,
            pltpu.SemaphoreType.DMA,
        ],
        compiler_params=pltpu.CompilerParams(
            use_tc_tiling_on_sc=False, needs_layout_passes=False
        ),
    )
    return f(xt, embt)


_BV = 1024
_K = _E + 1


def _mm_body(wb_ref, bowt_ref, out_ref):
    out_ref[...] = lax.dot_general(
        wb_ref[...],
        bowt_ref[...],
        dimension_numbers=(((0,), (0,)), ((), ())),
        preferred_element_type=jnp.float32,
    )


def _mm_call(wb, bowt1):
    return pl.pallas_call(
        _mm_body,
        grid=(pl.cdiv(_V, _BV),),
        in_specs=[
            pl.BlockSpec((_K, _BV), lambda j: (0, j)),
            pl.BlockSpec((_K, _B), lambda j: (0, 0)),
        ],
        out_specs=pl.BlockSpec((_BV, _B), lambda j: (j, 0)),
        out_shape=jax.ShapeDtypeStruct((_V, _B), jnp.float32),
        compiler_params=pltpu.CompilerParams(
            dimension_semantics=("arbitrary",),
        ),
    )(wb, bowt1)


def kernel(X, emb_table, W, b):
    xt = X.astype(jnp.int32).T.reshape(_L, _NCHUNK, _G).transpose(1, 0, 2)
    bowt = _bow_call(xt, emb_table.T)
    wb = jnp.concatenate([W.T, b[None, :]], axis=0)
    bowt1 = jnp.concatenate([bowt, jnp.ones((1, _B), jnp.float32)], axis=0)
    return _mm_call(wb, bowt1).T
